# baseline (speedup 1.0000x reference)
import functools

import jax
import jax.numpy as jnp
from jax import lax
from jax.experimental import pallas as pl
from jax.experimental.pallas import tpu as pltpu
from jax.experimental.pallas import tpu_sc as plsc

NC = 2
NS = 16
NW = NC * NS
CH = 80
BN = 1000


def _sc_mesh():
    return plsc.VectorSubcoreMesh(core_axis_name="c", subcore_axis_name="s")


def _zero_rows(ref, nrows, width):
    z = jnp.zeros((16,), jnp.float32)

    def row(i, carry):
        for j in range(width // 16):
            ref[i, pl.ds(j * 16, 16)] = z
        return carry

    lax.fori_loop(0, nrows, row, 0)


def _make_deg(n, e):
    nrows = e // 128
    mc = nrows // NW
    xtr = nrows - mc * NW
    zt = 10
    rpt = n // zt
    zr = 200

    @functools.partial(
        pl.kernel,
        mesh=_sc_mesh(),
        out_type=jax.ShapeDtypeStruct((NC * 2 * n, 16), jnp.float32),
        scratch_types=[
            pltpu.VMEM_SHARED((n, 16), jnp.float32),
            pltpu.VMEM_SHARED((n, 16), jnp.float32),
            pltpu.VMEM((mc + 1, 128), jnp.int32),
            pltpu.VMEM((mc + 1, 128), jnp.int32),
            pltpu.VMEM((128, 16), jnp.float32),
            pltpu.VMEM((zr, 16), jnp.float32),
            pltpu.SemaphoreType.DMA,
            pltpu.SemaphoreType.DMA,
        ],
        compiler_params=pltpu.CompilerParams(use_tc_tiling_on_sc=False),
    )
    def deg_kernel(src2_hbm, dst2_hbm, out_hbm, acc_s, acc_d, src_all, dst_all,
                   ones, zbuf, ssa, ssb):
        c = lax.axis_index("c")
        s = lax.axis_index("s")
        wid = s * NC + c

        one = jnp.full((16,), 1.0, jnp.float32)

        def fill_one(i, carry):
            ones[i, pl.ds(0, 16)] = one
            return carry

        lax.fori_loop(0, 128, fill_one, 0)
        _zero_rows(zbuf, zr, 16)

        @pl.when(s < zt)
        def _():
            for r in range(rpt // zr):
                pltpu.sync_copy(zbuf, acc_s.at[pl.ds(s * rpt + r * zr, zr)])
                pltpu.sync_copy(zbuf, acc_d.at[pl.ds(s * rpt + r * zr, zr)])

        rbase = wid * mc
        pltpu.sync_copy(src2_hbm.at[pl.ds(rbase, mc)], src_all.at[pl.ds(0, mc)])
        pltpu.sync_copy(dst2_hbm.at[pl.ds(rbase, mc)], dst_all.at[pl.ds(0, mc)])

        @pl.when(wid < xtr)
        def _():
            pltpu.sync_copy(src2_hbm.at[pl.ds(NW * mc + wid, 1)],
                            src_all.at[pl.ds(mc, 1)])
            pltpu.sync_copy(dst2_hbm.at[pl.ds(NW * mc + wid, 1)],
                            dst_all.at[pl.ds(mc, 1)])
        plsc.subcore_barrier()

        def chunk(ci, carry):
            @pl.when(ci > 0)
            def _():
                pltpu.make_async_copy(ones, acc_s.at[src_all.at[ci]], ssa).wait()
                pltpu.make_async_copy(ones, acc_d.at[dst_all.at[ci]], ssb).wait()
            pltpu.async_copy(ones, acc_s.at[src_all.at[ci]], ssa, add=True)
            pltpu.async_copy(ones, acc_d.at[dst_all.at[ci]], ssb, add=True)
            return carry

        lax.fori_loop(0, mc, chunk, 0)
        pltpu.make_async_copy(ones, acc_s.at[src_all.at[0]], ssa).wait()
        pltpu.make_async_copy(ones, acc_d.at[dst_all.at[0]], ssb).wait()

        @pl.when(wid < xtr)
        def _():
            pltpu.sync_copy(ones, acc_s.at[src_all.at[mc]], add=True)
            pltpu.sync_copy(ones, acc_d.at[dst_all.at[mc]], add=True)
        plsc.subcore_barrier()

        @pl.when(s < zt)
        def _():
            pltpu.sync_copy(acc_s.at[pl.ds(s * rpt, rpt)],
                            out_hbm.at[pl.ds(c * 2 * n + s * rpt, rpt)])
            pltpu.sync_copy(acc_d.at[pl.ds(s * rpt, rpt)],
                            out_hbm.at[pl.ds(c * 2 * n + n + s * rpt, rpt)])

    return deg_kernel


def _make_agg(n, e, d):
    nrows = e // 128
    mc = nrows // NW
    xtr = nrows - mc * NW
    zt = 10
    rpt = n // zt
    zr = 200

    @functools.partial(
        pl.kernel,
        mesh=_sc_mesh(),
        out_type=jax.ShapeDtypeStruct((NC * n, d), jnp.float32),
        scratch_types=[
            pltpu.VMEM_SHARED((n, d), jnp.float32),
            pltpu.VMEM((mc + 1, 128), jnp.int32),
            pltpu.VMEM((1, 128), jnp.int32),
            pltpu.VMEM((1, 128), jnp.int32),
            pltpu.VMEM((128, d), jnp.float32),
            pltpu.VMEM((128, d), jnp.float32),
            pltpu.SemaphoreType.DMA,
            pltpu.SemaphoreType.DMA,
            pltpu.SemaphoreType.DMA,
            pltpu.SemaphoreType.DMA,
        ],
        compiler_params=pltpu.CompilerParams(use_tc_tiling_on_sc=False),
    )
    def agg_kernel(sup_hbm, src2_hbm, dst2_hbm, out_hbm, acc,
                   dst_all, srcv0, srcv1, rows0, rows1, sem0, sem1, ssem0, ssem1):
        c = lax.axis_index("c")
        s = lax.axis_index("s")
        wid = s * NC + c
        rbase = wid * mc

        _zero_rows(rows0, 128, d)

        @pl.when(s < zt)
        def _():
            for r in range(rpt // 100):
                pltpu.sync_copy(rows0.at[pl.ds(0, 100)],
                                acc.at[pl.ds(s * rpt + r * 100, 100)])

        pltpu.sync_copy(dst2_hbm.at[pl.ds(rbase, mc)], dst_all.at[pl.ds(0, mc)])

        @pl.when(wid < xtr)
        def _():
            pltpu.sync_copy(dst2_hbm.at[pl.ds(NW * mc + wid, 1)],
                            dst_all.at[pl.ds(mc, 1)])
        plsc.subcore_barrier()

        pltpu.sync_copy(src2_hbm.at[pl.ds(rbase, 1)], srcv0)
        pltpu.sync_copy(src2_hbm.at[pl.ds(rbase + 1, 1)], srcv1)
        pltpu.async_copy(sup_hbm.at[srcv0.at[0]], rows0, sem0)
        pltpu.async_copy(sup_hbm.at[srcv1.at[0]], rows1, sem1)

        def gwait(rows, sem):
            pltpu.make_async_copy(sup_hbm.at[pl.ds(0, 128)], rows, sem).wait()

        def step(g, carry):
            c0 = 2 * g
            c1 = 2 * g + 1
            gwait(rows0, sem0)
            pltpu.async_copy(rows0, acc.at[dst_all.at[c0]], ssem0, add=True)
            pltpu.sync_copy(src2_hbm.at[pl.ds(rbase + c0 + 2, 1)], srcv0)
            gwait(rows1, sem1)
            pltpu.async_copy(rows1, acc.at[dst_all.at[c1]], ssem1, add=True)
            pltpu.sync_copy(src2_hbm.at[pl.ds(rbase + c1 + 2, 1)], srcv1)
            gwait(rows0, ssem0)
            pltpu.async_copy(sup_hbm.at[srcv0.at[0]], rows0, sem0)
            gwait(rows1, ssem1)
            pltpu.async_copy(sup_hbm.at[srcv1.at[0]], rows1, sem1)
            return carry

        lax.fori_loop(0, mc // 2 - 1, step, 0)

        gwait(rows0, sem0)
        pltpu.sync_copy(rows0, acc.at[dst_all.at[mc - 2]], add=True)
        gwait(rows1, sem1)
        pltpu.sync_copy(rows1, acc.at[dst_all.at[mc - 1]], add=True)

        @pl.when(wid < xtr)
        def _():
            pltpu.sync_copy(src2_hbm.at[pl.ds(NW * mc + wid, 1)], srcv0)
            pltpu.async_copy(sup_hbm.at[srcv0.at[0]], rows0, sem0).wait()
            pltpu.sync_copy(rows0, acc.at[dst_all.at[mc]], add=True)

        plsc.subcore_barrier()

        @pl.when(s < zt)
        def _():
            pltpu.sync_copy(acc.at[pl.ds(s * rpt, rpt)],
                            out_hbm.at[pl.ds(c * n + s * rpt, rpt)])

    return agg_kernel


def _row_spec(f):
    return pl.BlockSpec((BN, f), lambda i: (i, 0))


def _full_spec(shape):
    return pl.BlockSpec(shape, lambda i: tuple(0 for _ in shape))


def _half_spec(h, f):
    return pl.BlockSpec((1, BN, f), lambda i, h=h: (h, i, 0))


def _t0(x, dos0, dod0, dos1, dod1):
    n, f = x.shape

    def body(x_r, a0, b0, a1, b1, xs_r, ds_r, dd_r):
        dgo = a0[...] + a1[...]
        dgi = b0[...] + b1[...]
        dis = lax.rsqrt(jnp.maximum(dgo, 1.0))
        did = lax.rsqrt(jnp.maximum(dgi, 1.0))
        ds128 = jnp.broadcast_to(dis, (BN, 128))
        dd128 = jnp.broadcast_to(did, (BN, 128))
        xs_r[...] = x_r[...] * ds128
        ds_r[...] = ds128
        dd_r[...] = dd128

    dspec = pl.BlockSpec((BN, 1), lambda i: (i, 0))
    return pl.pallas_call(
        body,
        grid=(n // BN,),
        in_specs=[_row_spec(f), dspec, dspec, dspec, dspec],
        out_specs=[_row_spec(f), _row_spec(128), _row_spec(128)],
        out_shape=[jax.ShapeDtypeStruct((n, f), jnp.float32),
                   jax.ShapeDtypeStruct((n, 128), jnp.float32),
                   jax.ShapeDtypeStruct((n, 128), jnp.float32)],
    )(x, dos0, dod0, dos1, dod1)


def _t1(u1, dd, ds, W1, b1, W2):
    n = dd.shape[0]

    def body(ua, ub, dd_r, ds_r, w1, b1r, w2, s2a_r, s2b_r):
        u = (ua[0] + ub[0]) * dd_r[...]
        z = jnp.dot(u, w1[...], preferred_element_type=jnp.float32) + b1r[...]
        h = jnp.maximum(z, 0.0)
        s2 = jnp.dot(h, w2[...], preferred_element_type=jnp.float32)
        s2a_r[...] = s2[:, :128] * ds_r[...]
        s2b_r[...] = s2[:, 128:] * ds_r[...]

    return pl.pallas_call(
        body,
        grid=(n // BN,),
        in_specs=[_half_spec(0, 128), _half_spec(1, 128),
                  _row_spec(128), _row_spec(128),
                  _full_spec(W1.shape), _full_spec(b1.shape), _full_spec(W2.shape)],
        out_specs=[_row_spec(128), _row_spec(128)],
        out_shape=[jax.ShapeDtypeStruct((n, 128), jnp.float32),
                   jax.ShapeDtypeStruct((n, 128), jnp.float32)],
    )(u1, u1, dd, ds, W1, b1, W2)


def _t2(u2a, u2b, dd, ds, b2, W3):
    n = dd.shape[0]

    def body(ua0, ua1, ub0, ub1, dd_r, ds_r, b2r, w3, s3_r):
        ddv = dd_r[...]
        b2v = b2r[...]
        ha = jnp.maximum((ua0[0] + ua1[0]) * ddv + b2v[:, :128], 0.0)
        hb = jnp.maximum((ub0[0] + ub1[0]) * ddv + b2v[:, 128:], 0.0)
        w3v = w3[...]
        s3 = (jnp.dot(ha, w3v[:128, :], preferred_element_type=jnp.float32)
              + jnp.dot(hb, w3v[128:, :], preferred_element_type=jnp.float32))
        s3_r[...] = s3 * ds_r[...]

    return pl.pallas_call(
        body,
        grid=(n // BN,),
        in_specs=[_half_spec(0, 128), _half_spec(1, 128),
                  _half_spec(0, 128), _half_spec(1, 128),
                  _row_spec(128), _row_spec(128),
                  _full_spec(b2.shape), _full_spec(W3.shape)],
        out_specs=[_row_spec(128)],
        out_shape=[jax.ShapeDtypeStruct((n, 128), jnp.float32)],
    )(u2a, u2a, u2b, u2b, dd, ds, b2, W3)[0]


def _t3(u3, dd, ds, b3, W4):
    n = dd.shape[0]

    def body(ua, ub, dd_r, ds_r, b3r, w4, s4_r):
        h = jnp.maximum((ua[0] + ub[0]) * dd_r[...] + b3r[...], 0.0)
        s4 = jnp.dot(h, w4[...], preferred_element_type=jnp.float32)
        s4_r[...] = s4 * ds_r[...][:, :64]

    return pl.pallas_call(
        body,
        grid=(n // BN,),
        in_specs=[_half_spec(0, 128), _half_spec(1, 128),
                  _row_spec(128), _row_spec(128),
                  _full_spec(b3.shape), _full_spec(W4.shape)],
        out_specs=[_row_spec(64)],
        out_shape=[jax.ShapeDtypeStruct((n, 64), jnp.float32)],
    )(u3, u3, dd, ds, b3, W4)[0]


def _t4(u4, dd, b4):
    n = dd.shape[0]

    def body(ua, ub, dd_r, b4r, emb_r):
        emb_r[...] = (ua[0] + ub[0]) * dd_r[...][:, :64] + b4r[...]

    return pl.pallas_call(
        body,
        grid=(n // BN,),
        in_specs=[_half_spec(0, 64), _half_spec(1, 64),
                  _row_spec(128), _full_spec(b4.shape)],
        out_specs=[_row_spec(64)],
        out_shape=[jax.ShapeDtypeStruct((n, 64), jnp.float32)],
    )(u4, u4, dd, b4)[0]


def kernel(x, edge_index, W1, b1, W2, b2, W3, b3, W4, b4):
    n = x.shape[0]
    e = edge_index.shape[1]
    src = edge_index[0].astype(jnp.int32).reshape(e // 128, 128)
    dst = edge_index[1].astype(jnp.int32).reshape(e // 128, 128)

    deg = _make_deg(n, e)(src, dst)
    deg = deg[:, 0].reshape(NC, 2, n)
    dos0 = deg[0, 0].reshape(n, 1)
    dod0 = deg[0, 1].reshape(n, 1)
    dos1 = deg[1, 0].reshape(n, 1)
    dod1 = deg[1, 1].reshape(n, 1)
    xs, ds, dd = _t0(x, dos0, dod0, dos1, dod1)

    agg128 = _make_agg(n, e, 128)
    agg64 = _make_agg(n, e, 64)

    u1 = agg128(xs, src, dst).reshape(NC, n, 128)
    s2a, s2b = _t1(u1, dd, ds, W1, b1.reshape(1, -1), W2)

    u2a = agg128(s2a, src, dst).reshape(NC, n, 128)
    u2b = agg128(s2b, src, dst).reshape(NC, n, 128)
    s3 = _t2(u2a, u2b, dd, ds, b2.reshape(1, -1), W3)

    u3 = agg128(s3, src, dst).reshape(NC, n, 128)
    s4 = _t3(u3, dd, ds, b3.reshape(1, -1), W4)

    u4 = agg64(s4, src, dst).reshape(NC, n, 64)
    emb = _t4(u4, dd, b4.reshape(1, -1))
    return emb


# speedup vs baseline: 1.1229x; 1.1229x over previous
import functools

import jax
import jax.numpy as jnp
from jax import lax
from jax.experimental import pallas as pl
from jax.experimental.pallas import tpu as pltpu
from jax.experimental.pallas import tpu_sc as plsc

NC = 2
NS = 16
NW = NC * NS
CH = 80
BN = 1000


def _sc_mesh():
    return plsc.VectorSubcoreMesh(core_axis_name="c", subcore_axis_name="s")


def _zero_rows(ref, nrows, width):
    z = jnp.zeros((16,), jnp.float32)

    def row(i, carry):
        for j in range(width // 16):
            ref[i, pl.ds(j * 16, 16)] = z
        return carry

    lax.fori_loop(0, nrows, row, 0)


def _make_deg(n, e):
    nrows = e // 128
    mc = nrows // NW
    xtr = nrows - mc * NW
    zt = 10
    rpt = n // zt
    zr = 200

    @functools.partial(
        pl.kernel,
        mesh=_sc_mesh(),
        out_type=jax.ShapeDtypeStruct((NC * 2 * n, 16), jnp.float32),
        scratch_types=[
            pltpu.VMEM_SHARED((n, 16), jnp.float32),
            pltpu.VMEM_SHARED((n, 16), jnp.float32),
            pltpu.VMEM((mc + 1, 128), jnp.int32),
            pltpu.VMEM((mc + 1, 128), jnp.int32),
            pltpu.VMEM((128, 16), jnp.float32),
            pltpu.VMEM((zr, 16), jnp.float32),
            pltpu.SemaphoreType.DMA,
            pltpu.SemaphoreType.DMA,
        ],
        compiler_params=pltpu.CompilerParams(use_tc_tiling_on_sc=False),
    )
    def deg_kernel(src2_hbm, dst2_hbm, out_hbm, acc_s, acc_d, src_all, dst_all,
                   ones, zbuf, ssa, ssb):
        c = lax.axis_index("c")
        s = lax.axis_index("s")
        wid = s * NC + c

        one = jnp.full((16,), 1.0, jnp.float32)

        def fill_one(i, carry):
            ones[i, pl.ds(0, 16)] = one
            return carry

        lax.fori_loop(0, 128, fill_one, 0)
        _zero_rows(zbuf, zr, 16)

        @pl.when(s < zt)
        def _():
            for r in range(rpt // zr):
                pltpu.sync_copy(zbuf, acc_s.at[pl.ds(s * rpt + r * zr, zr)])
                pltpu.sync_copy(zbuf, acc_d.at[pl.ds(s * rpt + r * zr, zr)])

        rbase = wid * mc
        pltpu.sync_copy(src2_hbm.at[pl.ds(rbase, mc)], src_all.at[pl.ds(0, mc)])
        pltpu.sync_copy(dst2_hbm.at[pl.ds(rbase, mc)], dst_all.at[pl.ds(0, mc)])

        @pl.when(wid < xtr)
        def _():
            pltpu.sync_copy(src2_hbm.at[pl.ds(NW * mc + wid, 1)],
                            src_all.at[pl.ds(mc, 1)])
            pltpu.sync_copy(dst2_hbm.at[pl.ds(NW * mc + wid, 1)],
                            dst_all.at[pl.ds(mc, 1)])
        plsc.subcore_barrier()

        def chunk(ci, carry):
            @pl.when(ci > 0)
            def _():
                pltpu.make_async_copy(ones, acc_s.at[src_all.at[ci]], ssa).wait()
                pltpu.make_async_copy(ones, acc_d.at[dst_all.at[ci]], ssb).wait()
            pltpu.async_copy(ones, acc_s.at[src_all.at[ci]], ssa, add=True)
            pltpu.async_copy(ones, acc_d.at[dst_all.at[ci]], ssb, add=True)
            return carry

        lax.fori_loop(0, mc, chunk, 0)
        pltpu.make_async_copy(ones, acc_s.at[src_all.at[0]], ssa).wait()
        pltpu.make_async_copy(ones, acc_d.at[dst_all.at[0]], ssb).wait()

        @pl.when(wid < xtr)
        def _():
            pltpu.sync_copy(ones, acc_s.at[src_all.at[mc]], add=True)
            pltpu.sync_copy(ones, acc_d.at[dst_all.at[mc]], add=True)
        plsc.subcore_barrier()

        @pl.when(s < zt)
        def _():
            pltpu.sync_copy(acc_s.at[pl.ds(s * rpt, rpt)],
                            out_hbm.at[pl.ds(c * 2 * n + s * rpt, rpt)])
            pltpu.sync_copy(acc_d.at[pl.ds(s * rpt, rpt)],
                            out_hbm.at[pl.ds(c * 2 * n + n + s * rpt, rpt)])

    return deg_kernel


def _make_agg(n, e, d):
    nrows = e // 128
    mc = nrows // NW
    xtr = nrows - mc * NW
    zt = 10
    rpt = n // zt
    zr = 200

    @functools.partial(
        pl.kernel,
        mesh=_sc_mesh(),
        out_type=jax.ShapeDtypeStruct((NC * n, d), jnp.float32),
        scratch_types=[
            pltpu.VMEM_SHARED((n, d), jnp.float32),
            pltpu.VMEM((mc + 1, 128), jnp.int32),
            pltpu.VMEM((1, 128), jnp.int32),
            pltpu.VMEM((1, 128), jnp.int32),
            pltpu.VMEM((128, d), jnp.float32),
            pltpu.VMEM((128, d), jnp.float32),
            pltpu.SemaphoreType.DMA,
            pltpu.SemaphoreType.DMA,
        ],
        compiler_params=pltpu.CompilerParams(use_tc_tiling_on_sc=False),
    )
    def agg_kernel(sup_hbm, src2_hbm, dst2_hbm, out_hbm, acc,
                   dst_all, srcv0, srcv1, rows0, rows1, sem0, sem1):
        c = lax.axis_index("c")
        s = lax.axis_index("s")
        wid = s * NC + c
        rbase = wid * mc

        _zero_rows(rows0, 128, d)

        @pl.when(s < zt)
        def _():
            for r in range(rpt // 100):
                pltpu.sync_copy(rows0.at[pl.ds(0, 100)],
                                acc.at[pl.ds(s * rpt + r * 100, 100)])

        pltpu.sync_copy(dst2_hbm.at[pl.ds(rbase, mc)], dst_all.at[pl.ds(0, mc)])

        @pl.when(wid < xtr)
        def _():
            pltpu.sync_copy(dst2_hbm.at[pl.ds(NW * mc + wid, 1)],
                            dst_all.at[pl.ds(mc, 1)])
        plsc.subcore_barrier()

        pltpu.sync_copy(src2_hbm.at[pl.ds(rbase, 1)], srcv0)
        pltpu.sync_copy(src2_hbm.at[pl.ds(rbase + 1, 1)], srcv1)
        pltpu.async_copy(sup_hbm.at[srcv0.at[0]], rows0, sem0)
        pltpu.async_copy(sup_hbm.at[srcv1.at[0]], rows1, sem1)

        def gwait(rows, sem):
            pltpu.make_async_copy(sup_hbm.at[pl.ds(0, 128)], rows, sem).wait()

        def step(g, carry):
            c0 = 2 * g
            gwait(rows0, sem0)
            pltpu.sync_copy(rows0, acc.at[dst_all.at[c0]], add=True)
            pltpu.sync_copy(src2_hbm.at[pl.ds(rbase + c0 + 2, 1)], srcv0)
            pltpu.async_copy(sup_hbm.at[srcv0.at[0]], rows0, sem0)
            c1 = 2 * g + 1
            gwait(rows1, sem1)
            pltpu.sync_copy(rows1, acc.at[dst_all.at[c1]], add=True)
            pltpu.sync_copy(src2_hbm.at[pl.ds(rbase + c1 + 2, 1)], srcv1)
            pltpu.async_copy(sup_hbm.at[srcv1.at[0]], rows1, sem1)
            return carry

        lax.fori_loop(0, mc // 2 - 1, step, 0)

        gwait(rows0, sem0)
        pltpu.sync_copy(rows0, acc.at[dst_all.at[mc - 2]], add=True)
        gwait(rows1, sem1)
        pltpu.sync_copy(rows1, acc.at[dst_all.at[mc - 1]], add=True)

        @pl.when(wid < xtr)
        def _():
            pltpu.sync_copy(src2_hbm.at[pl.ds(NW * mc + wid, 1)], srcv0)
            pltpu.async_copy(sup_hbm.at[srcv0.at[0]], rows0, sem0).wait()
            pltpu.sync_copy(rows0, acc.at[dst_all.at[mc]], add=True)

        plsc.subcore_barrier()

        @pl.when(s < zt)
        def _():
            pltpu.sync_copy(acc.at[pl.ds(s * rpt, rpt)],
                            out_hbm.at[pl.ds(c * n + s * rpt, rpt)])

    return agg_kernel


def _row_spec(f):
    return pl.BlockSpec((BN, f), lambda i: (i, 0))


def _full_spec(shape):
    return pl.BlockSpec(shape, lambda i: tuple(0 for _ in shape))


def _half_spec(h, f):
    return pl.BlockSpec((1, BN, f), lambda i, h=h: (h, i, 0))


def _t0(x, dos0, dod0, dos1, dod1):
    n, f = x.shape

    def body(x_r, a0, b0, a1, b1, xs_r, ds_r, dd_r):
        dgo = a0[...] + a1[...]
        dgi = b0[...] + b1[...]
        dis = lax.rsqrt(jnp.maximum(dgo, 1.0))
        did = lax.rsqrt(jnp.maximum(dgi, 1.0))
        ds128 = jnp.broadcast_to(dis, (BN, 128))
        dd128 = jnp.broadcast_to(did, (BN, 128))
        xs_r[...] = x_r[...] * ds128
        ds_r[...] = ds128
        dd_r[...] = dd128

    dspec = pl.BlockSpec((BN, 1), lambda i: (i, 0))
    return pl.pallas_call(
        body,
        grid=(n // BN,),
        in_specs=[_row_spec(f), dspec, dspec, dspec, dspec],
        out_specs=[_row_spec(f), _row_spec(128), _row_spec(128)],
        out_shape=[jax.ShapeDtypeStruct((n, f), jnp.float32),
                   jax.ShapeDtypeStruct((n, 128), jnp.float32),
                   jax.ShapeDtypeStruct((n, 128), jnp.float32)],
    )(x, dos0, dod0, dos1, dod1)


def _t1(u1, dd, ds, W1, b1, W2):
    n = dd.shape[0]

    def body(ua, ub, dd_r, ds_r, w1, b1r, w2, s2a_r, s2b_r):
        u = (ua[0] + ub[0]) * dd_r[...]
        z = jnp.dot(u, w1[...], preferred_element_type=jnp.float32) + b1r[...]
        h = jnp.maximum(z, 0.0)
        s2 = jnp.dot(h, w2[...], preferred_element_type=jnp.float32)
        s2a_r[...] = s2[:, :128] * ds_r[...]
        s2b_r[...] = s2[:, 128:] * ds_r[...]

    return pl.pallas_call(
        body,
        grid=(n // BN,),
        in_specs=[_half_spec(0, 128), _half_spec(1, 128),
                  _row_spec(128), _row_spec(128),
                  _full_spec(W1.shape), _full_spec(b1.shape), _full_spec(W2.shape)],
        out_specs=[_row_spec(128), _row_spec(128)],
        out_shape=[jax.ShapeDtypeStruct((n, 128), jnp.float32),
                   jax.ShapeDtypeStruct((n, 128), jnp.float32)],
    )(u1, u1, dd, ds, W1, b1, W2)


def _t2(u2a, u2b, dd, ds, b2, W3):
    n = dd.shape[0]

    def body(ua0, ua1, ub0, ub1, dd_r, ds_r, b2r, w3, s3_r):
        ddv = dd_r[...]
        b2v = b2r[...]
        ha = jnp.maximum((ua0[0] + ua1[0]) * ddv + b2v[:, :128], 0.0)
        hb = jnp.maximum((ub0[0] + ub1[0]) * ddv + b2v[:, 128:], 0.0)
        w3v = w3[...]
        s3 = (jnp.dot(ha, w3v[:128, :], preferred_element_type=jnp.float32)
              + jnp.dot(hb, w3v[128:, :], preferred_element_type=jnp.float32))
        s3_r[...] = s3 * ds_r[...]

    return pl.pallas_call(
        body,
        grid=(n // BN,),
        in_specs=[_half_spec(0, 128), _half_spec(1, 128),
                  _half_spec(0, 128), _half_spec(1, 128),
                  _row_spec(128), _row_spec(128),
                  _full_spec(b2.shape), _full_spec(W3.shape)],
        out_specs=[_row_spec(128)],
        out_shape=[jax.ShapeDtypeStruct((n, 128), jnp.float32)],
    )(u2a, u2a, u2b, u2b, dd, ds, b2, W3)[0]


def _t3(u3, dd, ds, b3, W4):
    n = dd.shape[0]

    def body(ua, ub, dd_r, ds_r, b3r, w4, s4_r):
        h = jnp.maximum((ua[0] + ub[0]) * dd_r[...] + b3r[...], 0.0)
        s4 = jnp.dot(h, w4[...], preferred_element_type=jnp.float32)
        s4_r[...] = s4 * ds_r[...][:, :64]

    return pl.pallas_call(
        body,
        grid=(n // BN,),
        in_specs=[_half_spec(0, 128), _half_spec(1, 128),
                  _row_spec(128), _row_spec(128),
                  _full_spec(b3.shape), _full_spec(W4.shape)],
        out_specs=[_row_spec(64)],
        out_shape=[jax.ShapeDtypeStruct((n, 64), jnp.float32)],
    )(u3, u3, dd, ds, b3, W4)[0]


def _t4(u4, dd, b4):
    n = dd.shape[0]

    def body(ua, ub, dd_r, b4r, emb_r):
        emb_r[...] = (ua[0] + ub[0]) * dd_r[...][:, :64] + b4r[...]

    return pl.pallas_call(
        body,
        grid=(n // BN,),
        in_specs=[_half_spec(0, 64), _half_spec(1, 64),
                  _row_spec(128), _full_spec(b4.shape)],
        out_specs=[_row_spec(64)],
        out_shape=[jax.ShapeDtypeStruct((n, 64), jnp.float32)],
    )(u4, u4, dd, b4)[0]


def kernel(x, edge_index, W1, b1, W2, b2, W3, b3, W4, b4):
    n = x.shape[0]
    e = edge_index.shape[1]
    src = edge_index[0].astype(jnp.int32).reshape(e // 128, 128)
    dst = edge_index[1].astype(jnp.int32).reshape(e // 128, 128)

    deg = _make_deg(n, e)(src, dst)
    deg = deg[:, 0].reshape(NC, 2, n)
    dos0 = deg[0, 0].reshape(n, 1)
    dod0 = deg[0, 1].reshape(n, 1)
    dos1 = deg[1, 0].reshape(n, 1)
    dod1 = deg[1, 1].reshape(n, 1)
    xs, ds, dd = _t0(x, dos0, dod0, dos1, dod1)

    agg128 = _make_agg(n, e, 128)
    agg64 = _make_agg(n, e, 64)

    u1 = agg128(xs, src, dst).reshape(NC, n, 128)
    s2a, s2b = _t1(u1, dd, ds, W1, b1.reshape(1, -1), W2)

    u2a = agg128(s2a, src, dst).reshape(NC, n, 128)
    u2b = agg128(s2b, src, dst).reshape(NC, n, 128)
    s3 = _t2(u2a, u2b, dd, ds, b2.reshape(1, -1), W3)

    u3 = agg128(s3, src, dst).reshape(NC, n, 128)
    s4 = _t3(u3, dd, ds, b3.reshape(1, -1), W4)

    u4 = agg64(s4, src, dst).reshape(NC, n, 64)
    emb = _t4(u4, dd, b4.reshape(1, -1))
    return emb


# speedup vs baseline: 1.1267x; 1.0034x over previous
import functools

import jax
import jax.numpy as jnp
from jax import lax
from jax.experimental import pallas as pl
from jax.experimental.pallas import tpu as pltpu
from jax.experimental.pallas import tpu_sc as plsc

NC = 2
NS = 16
NW = NC * NS
CH = 80
BN = 1000


def _sc_mesh():
    return plsc.VectorSubcoreMesh(core_axis_name="c", subcore_axis_name="s")


def _zero_rows(ref, nrows, width):
    z = jnp.zeros((16,), jnp.float32)

    def row(i, carry):
        for j in range(width // 16):
            ref[i, pl.ds(j * 16, 16)] = z
        return carry

    lax.fori_loop(0, nrows, row, 0)


def _make_deg(n, e):
    ept = e // NW
    cw = 512
    nfull = ept // cw
    rem = ept - nfull * cw
    zt = 10
    rpt = n // zt
    zr = 200

    @functools.partial(
        pl.kernel,
        mesh=_sc_mesh(),
        out_type=jax.ShapeDtypeStruct((NC * 2 * n, 16), jnp.float32),
        scratch_types=[
            pltpu.VMEM_SHARED((n, 16), jnp.float32),
            pltpu.VMEM_SHARED((n, 16), jnp.float32),
            pltpu.VMEM((ept,), jnp.int32),
            pltpu.VMEM((ept,), jnp.int32),
            pltpu.VMEM((cw, 16), jnp.float32),
            pltpu.VMEM((zr, 16), jnp.float32),
            pltpu.SemaphoreType.DMA,
            pltpu.SemaphoreType.DMA,
        ],
        compiler_params=pltpu.CompilerParams(use_tc_tiling_on_sc=False),
    )
    def deg_kernel(src_hbm, dst_hbm, out_hbm, acc_s, acc_d, src_st, dst_st,
                   ones, zbuf, ssa, ssb):
        c = lax.axis_index("c")
        s = lax.axis_index("s")
        wid = s * NC + c

        one = jnp.full((16,), 1.0, jnp.float32)

        def fill_one(i, carry):
            ones[i, pl.ds(0, 16)] = one
            return carry

        lax.fori_loop(0, cw, fill_one, 0)
        _zero_rows(zbuf, zr, 16)

        @pl.when(s < zt)
        def _():
            for r in range(rpt // zr):
                pltpu.sync_copy(zbuf, acc_s.at[pl.ds(s * rpt + r * zr, zr)])
                pltpu.sync_copy(zbuf, acc_d.at[pl.ds(s * rpt + r * zr, zr)])

        ebase = wid * ept
        pltpu.sync_copy(src_hbm.at[pl.ds(ebase, ept)], src_st)
        pltpu.sync_copy(dst_hbm.at[pl.ds(ebase, ept)], dst_st)
        plsc.subcore_barrier()

        def chunk(ci, carry):
            @pl.when(ci > 0)
            def _():
                pltpu.make_async_copy(ones, acc_s.at[src_st.at[pl.ds(ci * cw, cw)]], ssa).wait()
                pltpu.make_async_copy(ones, acc_d.at[dst_st.at[pl.ds(ci * cw, cw)]], ssb).wait()
            pltpu.async_copy(ones, acc_s.at[src_st.at[pl.ds(ci * cw, cw)]], ssa, add=True)
            pltpu.async_copy(ones, acc_d.at[dst_st.at[pl.ds(ci * cw, cw)]], ssb, add=True)
            return carry

        lax.fori_loop(0, nfull, chunk, 0)
        pltpu.make_async_copy(ones, acc_s.at[src_st.at[pl.ds(0, cw)]], ssa).wait()
        pltpu.make_async_copy(ones, acc_d.at[dst_st.at[pl.ds(0, cw)]], ssb).wait()

        if rem:
            pltpu.sync_copy(ones.at[pl.ds(0, rem)],
                            acc_s.at[src_st.at[pl.ds(nfull * cw, rem)]], add=True)
            pltpu.sync_copy(ones.at[pl.ds(0, rem)],
                            acc_d.at[dst_st.at[pl.ds(nfull * cw, rem)]], add=True)
        plsc.subcore_barrier()

        @pl.when(s < zt)
        def _():
            pltpu.sync_copy(acc_s.at[pl.ds(s * rpt, rpt)],
                            out_hbm.at[pl.ds(c * 2 * n + s * rpt, rpt)])
            pltpu.sync_copy(acc_d.at[pl.ds(s * rpt, rpt)],
                            out_hbm.at[pl.ds(c * 2 * n + n + s * rpt, rpt)])

    return deg_kernel


def _make_agg(n, e, d):
    nrows = e // 128
    mc = nrows // NW
    xtr = nrows - mc * NW
    zt = 10
    rpt = n // zt
    zr = 200

    @functools.partial(
        pl.kernel,
        mesh=_sc_mesh(),
        out_type=jax.ShapeDtypeStruct((NC * n, d), jnp.float32),
        scratch_types=[
            pltpu.VMEM_SHARED((n, d), jnp.float32),
            pltpu.VMEM((mc + 1, 128), jnp.int32),
            pltpu.VMEM((1, 128), jnp.int32),
            pltpu.VMEM((1, 128), jnp.int32),
            pltpu.VMEM((128, d), jnp.float32),
            pltpu.VMEM((128, d), jnp.float32),
            pltpu.SemaphoreType.DMA,
            pltpu.SemaphoreType.DMA,
        ],
        compiler_params=pltpu.CompilerParams(use_tc_tiling_on_sc=False),
    )
    def agg_kernel(sup_hbm, src2_hbm, dst2_hbm, out_hbm, acc,
                   dst_all, srcv0, srcv1, rows0, rows1, sem0, sem1):
        c = lax.axis_index("c")
        s = lax.axis_index("s")
        wid = s * NC + c
        rbase = wid * mc

        _zero_rows(rows0, 128, d)

        @pl.when(s < zt)
        def _():
            for r in range(rpt // 100):
                pltpu.sync_copy(rows0.at[pl.ds(0, 100)],
                                acc.at[pl.ds(s * rpt + r * 100, 100)])

        pltpu.sync_copy(dst2_hbm.at[pl.ds(rbase, mc)], dst_all.at[pl.ds(0, mc)])

        @pl.when(wid < xtr)
        def _():
            pltpu.sync_copy(dst2_hbm.at[pl.ds(NW * mc + wid, 1)],
                            dst_all.at[pl.ds(mc, 1)])
        plsc.subcore_barrier()

        pltpu.sync_copy(src2_hbm.at[pl.ds(rbase, 1)], srcv0)
        pltpu.sync_copy(src2_hbm.at[pl.ds(rbase + 1, 1)], srcv1)
        pltpu.async_copy(sup_hbm.at[srcv0.at[0]], rows0, sem0)
        pltpu.async_copy(sup_hbm.at[srcv1.at[0]], rows1, sem1)

        def gwait(rows, sem):
            pltpu.make_async_copy(sup_hbm.at[pl.ds(0, 128)], rows, sem).wait()

        def step(g, carry):
            c0 = 2 * g
            gwait(rows0, sem0)
            pltpu.sync_copy(rows0, acc.at[dst_all.at[c0]], add=True)
            pltpu.sync_copy(src2_hbm.at[pl.ds(rbase + c0 + 2, 1)], srcv0)
            pltpu.async_copy(sup_hbm.at[srcv0.at[0]], rows0, sem0)
            c1 = 2 * g + 1
            gwait(rows1, sem1)
            pltpu.sync_copy(rows1, acc.at[dst_all.at[c1]], add=True)
            pltpu.sync_copy(src2_hbm.at[pl.ds(rbase + c1 + 2, 1)], srcv1)
            pltpu.async_copy(sup_hbm.at[srcv1.at[0]], rows1, sem1)
            return carry

        lax.fori_loop(0, mc // 2 - 1, step, 0)

        gwait(rows0, sem0)
        pltpu.sync_copy(rows0, acc.at[dst_all.at[mc - 2]], add=True)
        gwait(rows1, sem1)
        pltpu.sync_copy(rows1, acc.at[dst_all.at[mc - 1]], add=True)

        @pl.when(wid < xtr)
        def _():
            pltpu.sync_copy(src2_hbm.at[pl.ds(NW * mc + wid, 1)], srcv0)
            pltpu.async_copy(sup_hbm.at[srcv0.at[0]], rows0, sem0).wait()
            pltpu.sync_copy(rows0, acc.at[dst_all.at[mc]], add=True)

        plsc.subcore_barrier()

        @pl.when(s < zt)
        def _():
            pltpu.sync_copy(acc.at[pl.ds(s * rpt, rpt)],
                            out_hbm.at[pl.ds(c * n + s * rpt, rpt)])

    return agg_kernel


def _row_spec(f):
    return pl.BlockSpec((BN, f), lambda i: (i, 0))


def _full_spec(shape):
    return pl.BlockSpec(shape, lambda i: tuple(0 for _ in shape))


def _half_spec(h, f):
    return pl.BlockSpec((1, BN, f), lambda i, h=h: (h, i, 0))


def _t0(x, dos0, dod0, dos1, dod1):
    n, f = x.shape

    def body(x_r, a0, b0, a1, b1, xs_r, ds_r, dd_r):
        dgo = a0[...] + a1[...]
        dgi = b0[...] + b1[...]
        dis = lax.rsqrt(jnp.maximum(dgo, 1.0))
        did = lax.rsqrt(jnp.maximum(dgi, 1.0))
        ds128 = jnp.broadcast_to(dis, (BN, 128))
        dd128 = jnp.broadcast_to(did, (BN, 128))
        xs_r[...] = x_r[...] * ds128
        ds_r[...] = ds128
        dd_r[...] = dd128

    dspec = pl.BlockSpec((BN, 1), lambda i: (i, 0))
    return pl.pallas_call(
        body,
        grid=(n // BN,),
        in_specs=[_row_spec(f), dspec, dspec, dspec, dspec],
        out_specs=[_row_spec(f), _row_spec(128), _row_spec(128)],
        out_shape=[jax.ShapeDtypeStruct((n, f), jnp.float32),
                   jax.ShapeDtypeStruct((n, 128), jnp.float32),
                   jax.ShapeDtypeStruct((n, 128), jnp.float32)],
    )(x, dos0, dod0, dos1, dod1)


def _t1(u1, dd, ds, W1, b1, W2):
    n = dd.shape[0]

    def body(ua, ub, dd_r, ds_r, w1, b1r, w2, s2a_r, s2b_r):
        u = (ua[0] + ub[0]) * dd_r[...]
        z = jnp.dot(u, w1[...], preferred_element_type=jnp.float32) + b1r[...]
        h = jnp.maximum(z, 0.0)
        s2 = jnp.dot(h, w2[...], preferred_element_type=jnp.float32)
        s2a_r[...] = s2[:, :128] * ds_r[...]
        s2b_r[...] = s2[:, 128:] * ds_r[...]

    return pl.pallas_call(
        body,
        grid=(n // BN,),
        in_specs=[_half_spec(0, 128), _half_spec(1, 128),
                  _row_spec(128), _row_spec(128),
                  _full_spec(W1.shape), _full_spec(b1.shape), _full_spec(W2.shape)],
        out_specs=[_row_spec(128), _row_spec(128)],
        out_shape=[jax.ShapeDtypeStruct((n, 128), jnp.float32),
                   jax.ShapeDtypeStruct((n, 128), jnp.float32)],
    )(u1, u1, dd, ds, W1, b1, W2)


def _t2(u2a, u2b, dd, ds, b2, W3):
    n = dd.shape[0]

    def body(ua0, ua1, ub0, ub1, dd_r, ds_r, b2r, w3, s3_r):
        ddv = dd_r[...]
        b2v = b2r[...]
        ha = jnp.maximum((ua0[0] + ua1[0]) * ddv + b2v[:, :128], 0.0)
        hb = jnp.maximum((ub0[0] + ub1[0]) * ddv + b2v[:, 128:], 0.0)
        w3v = w3[...]
        s3 = (jnp.dot(ha, w3v[:128, :], preferred_element_type=jnp.float32)
              + jnp.dot(hb, w3v[128:, :], preferred_element_type=jnp.float32))
        s3_r[...] = s3 * ds_r[...]

    return pl.pallas_call(
        body,
        grid=(n // BN,),
        in_specs=[_half_spec(0, 128), _half_spec(1, 128),
                  _half_spec(0, 128), _half_spec(1, 128),
                  _row_spec(128), _row_spec(128),
                  _full_spec(b2.shape), _full_spec(W3.shape)],
        out_specs=[_row_spec(128)],
        out_shape=[jax.ShapeDtypeStruct((n, 128), jnp.float32)],
    )(u2a, u2a, u2b, u2b, dd, ds, b2, W3)[0]


def _t3(u3, dd, ds, b3, W4):
    n = dd.shape[0]

    def body(ua, ub, dd_r, ds_r, b3r, w4, s4_r):
        h = jnp.maximum((ua[0] + ub[0]) * dd_r[...] + b3r[...], 0.0)
        s4 = jnp.dot(h, w4[...], preferred_element_type=jnp.float32)
        s4_r[...] = s4 * ds_r[...][:, :64]

    return pl.pallas_call(
        body,
        grid=(n // BN,),
        in_specs=[_half_spec(0, 128), _half_spec(1, 128),
                  _row_spec(128), _row_spec(128),
                  _full_spec(b3.shape), _full_spec(W4.shape)],
        out_specs=[_row_spec(64)],
        out_shape=[jax.ShapeDtypeStruct((n, 64), jnp.float32)],
    )(u3, u3, dd, ds, b3, W4)[0]


def _t4(u4, dd, b4):
    n = dd.shape[0]

    def body(ua, ub, dd_r, b4r, emb_r):
        emb_r[...] = (ua[0] + ub[0]) * dd_r[...][:, :64] + b4r[...]

    return pl.pallas_call(
        body,
        grid=(n // BN,),
        in_specs=[_half_spec(0, 64), _half_spec(1, 64),
                  _row_spec(128), _full_spec(b4.shape)],
        out_specs=[_row_spec(64)],
        out_shape=[jax.ShapeDtypeStruct((n, 64), jnp.float32)],
    )(u4, u4, dd, b4)[0]


def kernel(x, edge_index, W1, b1, W2, b2, W3, b3, W4, b4):
    n = x.shape[0]
    e = edge_index.shape[1]
    src_flat = edge_index[0].astype(jnp.int32)
    dst_flat = edge_index[1].astype(jnp.int32)
    src = src_flat.reshape(e // 128, 128)
    dst = dst_flat.reshape(e // 128, 128)

    deg = _make_deg(n, e)(src_flat, dst_flat)
    deg = deg[:, 0].reshape(NC, 2, n)
    dos0 = deg[0, 0].reshape(n, 1)
    dod0 = deg[0, 1].reshape(n, 1)
    dos1 = deg[1, 0].reshape(n, 1)
    dod1 = deg[1, 1].reshape(n, 1)
    xs, ds, dd = _t0(x, dos0, dod0, dos1, dod1)

    agg128 = _make_agg(n, e, 128)
    agg64 = _make_agg(n, e, 64)

    u1 = agg128(xs, src, dst).reshape(NC, n, 128)
    s2a, s2b = _t1(u1, dd, ds, W1, b1.reshape(1, -1), W2)

    u2a = agg128(s2a, src, dst).reshape(NC, n, 128)
    u2b = agg128(s2b, src, dst).reshape(NC, n, 128)
    s3 = _t2(u2a, u2b, dd, ds, b2.reshape(1, -1), W3)

    u3 = agg128(s3, src, dst).reshape(NC, n, 128)
    s4 = _t3(u3, dd, ds, b3.reshape(1, -1), W4)

    u4 = agg64(s4, src, dst).reshape(NC, n, 64)
    emb = _t4(u4, dd, b4.reshape(1, -1))
    return emb
